# Initial kernel scaffold; baseline (speedup 1.0000x reference)
import jax
import jax.numpy as jnp
from jax.experimental import pallas as pl


def kernel(x, enc_W, enc_b, dec_W):
    raise NotImplementedError("write your pallas kernel here")


# speedup vs baseline: 3.0386x; 3.0386x over previous
import functools

import jax
import jax.numpy as jnp
from jax.experimental import pallas as pl

GROUPS = 8
SUB = 8192
DM = 768
TOPK = 32
ENC_BLK = 2048
DEC_BLK = 2048


def _monotonic_key(v):
    b = jax.lax.bitcast_convert_type(v, jnp.int32)
    flip = jax.lax.shift_right_arithmetic(b, 31) & jnp.int32(0x7FFFFFFF)
    return b ^ flip


def _topk_mask(pre, k):
    rows = pre.shape[0]
    key = _monotonic_key(pre)

    def count_ge(t):
        return jnp.sum((key >= t).astype(jnp.int32), axis=1, keepdims=True)

    t = jnp.where(count_ge(jnp.zeros((rows, 1), jnp.int32)) >= k,
                  jnp.int32(0), jnp.int32(-2147483648))
    t = jnp.broadcast_to(t, (rows, 1))

    def body(i, t):
        cand = t | (jnp.int32(1) << (30 - i))
        return jnp.where(count_ge(cand) >= k, cand, t)

    t = jax.lax.fori_loop(0, 31, body, t)

    n_gt = jnp.sum((key > t).astype(jnp.int32), axis=1, keepdims=True)
    m = k - n_gt
    tie = key == t
    col = jax.lax.broadcasted_iota(jnp.int32, pre.shape, 1)

    def tie_body(i, p):
        cand = p | (jnp.int32(1) << (12 - i))
        cnt = jnp.sum((tie & (col < cand)).astype(jnp.int32), axis=1,
                      keepdims=True)
        return jnp.where(cnt < m, cand, p)

    p = jax.lax.fori_loop(0, 13, tie_body, jnp.zeros((rows, 1), jnp.int32))
    p = jnp.where(m >= 1, p, jnp.int32(-1))
    return (key > t) | (tie & (col <= p))


def _encode_body(x_ref, w_ref, b_ref, out_ref):
    j = pl.program_id(1)
    nb = pl.num_programs(1)
    pre = jax.lax.dot_general(
        x_ref[...], w_ref[0],
        dimension_numbers=(((1,), (1,)), ((), ())),
        preferred_element_type=jnp.float32,
    ) + b_ref[0]
    out_ref[:, pl.ds(j * ENC_BLK, ENC_BLK)] = pre

    @pl.when(j == nb - 1)
    def _():
        full = out_ref[...]
        mask = _topk_mask(full, TOPK)
        out_ref[...] = jnp.where(mask, jnp.maximum(full, 0.0), 0.0)


def _decode_body(a_ref, w_ref, o_ref):
    g = pl.program_id(0)
    j = pl.program_id(1)

    @pl.when((g == 0) & (j == 0))
    def _():
        o_ref[...] = jnp.zeros_like(o_ref)

    o_ref[...] += jax.lax.dot_general(
        a_ref[...], w_ref[0],
        dimension_numbers=(((1,), (1,)), ((), ())),
        preferred_element_type=jnp.float32,
    )


@jax.jit
def kernel(x, enc_W, enc_b, dec_W):
    batch = x.shape[0]
    nb_enc = SUB // ENC_BLK
    nb_dec = SUB // DEC_BLK
    enc_b3 = enc_b.reshape(GROUPS, 1, SUB)

    full_acts = pl.pallas_call(
        _encode_body,
        grid=(GROUPS, nb_enc),
        in_specs=[
            pl.BlockSpec((batch, DM), lambda g, j: (0, 0)),
            pl.BlockSpec((1, ENC_BLK, DM), lambda g, j: (g, j, 0)),
            pl.BlockSpec((1, 1, ENC_BLK), lambda g, j: (g, 0, j)),
        ],
        out_specs=pl.BlockSpec((batch, SUB), lambda g, j: (0, g)),
        out_shape=jax.ShapeDtypeStruct((batch, GROUPS * SUB), jnp.float32),
    )(x, enc_W, enc_b3)

    final_recon = pl.pallas_call(
        _decode_body,
        grid=(GROUPS, nb_dec),
        in_specs=[
            pl.BlockSpec((batch, DEC_BLK),
                         lambda g, j: (0, g * (SUB // DEC_BLK) + j)),
            pl.BlockSpec((1, DM, DEC_BLK), lambda g, j: (g, 0, j)),
        ],
        out_specs=pl.BlockSpec((batch, DM), lambda g, j: (0, 0)),
        out_shape=jax.ShapeDtypeStruct((batch, DM), jnp.float32),
    )(full_acts, dec_W)

    return (final_recon, full_acts)


# speedup vs baseline: 3.5040x; 1.1532x over previous
import jax
import jax.numpy as jnp
from jax.experimental import pallas as pl
from jax.experimental.pallas import tpu as pltpu

GROUPS = 8
SUB = 8192
DM = 768
TOPK = 32
ENC_BLK = 2048
NB = SUB // ENC_BLK
DEC_BLK = 2048


def _monotonic_key(v):
    b = jax.lax.bitcast_convert_type(v, jnp.int32)
    flip = jax.lax.shift_right_arithmetic(b, 31) & jnp.int32(0x7FFFFFFF)
    return b ^ flip


def _count_ge(key, t):
    return jnp.sum((key >= t).astype(jnp.int32), axis=1, keepdims=True)


def _descend_bits(key, t, start, n):
    def body(i, t):
        cand = t | (jnp.int32(1) << (start - i))
        return jnp.where(_count_ge(key, cand) >= TOPK, cand, t)
    return jax.lax.fori_loop(0, n, body, t)


def _select_acts(key, t, p_ref):
    n_ge = _count_ge(key, t)
    col = jax.lax.broadcasted_iota(jnp.int32, key.shape, 1)
    p_ref[...] = jnp.full(p_ref.shape, jnp.int32(8191))

    @pl.when(jnp.any(n_ge > TOPK))
    def _():
        n_gt = jnp.sum((key > t).astype(jnp.int32), axis=1, keepdims=True)
        m = TOPK - n_gt
        tie = key == t

        def body(i, p):
            cand = p | (jnp.int32(1) << (12 - i))
            cnt = jnp.sum((tie & (col < cand)).astype(jnp.int32), axis=1,
                          keepdims=True)
            return jnp.where(cnt < m, cand, p)

        p = jax.lax.fori_loop(0, 13, body,
                              jnp.zeros((key.shape[0], 1), jnp.int32))
        p_ref[...] = jnp.broadcast_to(p, p_ref.shape)

    sel = (key > t) | ((key == t) & (col <= p_ref[:, 0:1]))
    return jnp.where(sel & (key > 0),
                     jax.lax.bitcast_convert_type(key, jnp.float32), 0.0)


def _encode_body(x_ref, w_ref, b_ref, out_ref, mk_ref, t_ref, p_ref):
    g = pl.program_id(0)
    j = pl.program_id(1)

    @pl.when(g < GROUPS)
    def _():
        pre = jax.lax.dot_general(
            x_ref[...], w_ref[0],
            dimension_numbers=(((1,), (1,)), ((), ())),
            preferred_element_type=jnp.float32,
        ) + b_ref[0]
        par = jax.lax.rem(g, 2)
        mk_ref[par, :, pl.ds(j * ENC_BLK, ENC_BLK)] = _monotonic_key(pre)

    @pl.when(g > 0)
    def _():
        key = mk_ref[jax.lax.rem(g - 1, 2)]
        rows = key.shape[0]

        @pl.when(j == 0)
        def _():
            zero = jnp.zeros((rows, 1), jnp.int32)
            t = jnp.where(_count_ge(key, zero) >= TOPK,
                          zero, jnp.full((rows, 1), jnp.int32(-2147483648)))
            t = _descend_bits(key, t, 30, 9)
            t_ref[...] = jnp.broadcast_to(t, t_ref.shape)

        @pl.when(j == 1)
        def _():
            t = _descend_bits(key, t_ref[:, 0:1], 21, 11)
            t_ref[...] = jnp.broadcast_to(t, t_ref.shape)

        @pl.when(j == 2)
        def _():
            t = _descend_bits(key, t_ref[:, 0:1], 10, 11)
            t_ref[...] = jnp.broadcast_to(t, t_ref.shape)

        @pl.when(j == 3)
        def _():
            out_ref[...] = _select_acts(key, t_ref[:, 0:1], p_ref)


def _decode_body(a_ref, w_ref, o_ref):
    g = pl.program_id(0)
    j = pl.program_id(1)

    @pl.when((g == 0) & (j == 0))
    def _():
        o_ref[...] = jnp.zeros_like(o_ref)

    o_ref[...] += jax.lax.dot_general(
        a_ref[...], w_ref[0],
        dimension_numbers=(((1,), (1,)), ((), ())),
        preferred_element_type=jnp.float32,
    )


@jax.jit
def kernel(x, enc_W, enc_b, dec_W):
    batch = x.shape[0]
    nb_dec = SUB // DEC_BLK
    enc_b3 = enc_b.reshape(GROUPS, 1, SUB)

    full_acts = pl.pallas_call(
        _encode_body,
        grid=(GROUPS + 1, NB),
        in_specs=[
            pl.BlockSpec((batch, DM), lambda g, j: (0, 0)),
            pl.BlockSpec((1, ENC_BLK, DM),
                         lambda g, j: (jnp.minimum(g, GROUPS - 1),
                                       jnp.where(g < GROUPS, j, NB - 1), 0)),
            pl.BlockSpec((1, 1, ENC_BLK),
                         lambda g, j: (jnp.minimum(g, GROUPS - 1), 0,
                                       jnp.where(g < GROUPS, j, NB - 1))),
        ],
        out_specs=pl.BlockSpec((batch, SUB),
                               lambda g, j: (0, jnp.maximum(g - 1, 0))),
        out_shape=jax.ShapeDtypeStruct((batch, GROUPS * SUB), jnp.float32),
        scratch_shapes=[
            pltpu.VMEM((2, batch, SUB), jnp.int32),
            pltpu.VMEM((batch, 128), jnp.int32),
            pltpu.VMEM((batch, 128), jnp.int32),
        ],
    )(x, enc_W, enc_b3)

    final_recon = pl.pallas_call(
        _decode_body,
        grid=(GROUPS, nb_dec),
        in_specs=[
            pl.BlockSpec((batch, DEC_BLK),
                         lambda g, j: (0, g * (SUB // DEC_BLK) + j)),
            pl.BlockSpec((1, DM, DEC_BLK), lambda g, j: (g, 0, j)),
        ],
        out_specs=pl.BlockSpec((batch, DM), lambda g, j: (0, 0)),
        out_shape=jax.ShapeDtypeStruct((batch, DM), jnp.float32),
    )(full_acts, dec_W)

    return (final_recon, full_acts)
